# baseline (speedup 1.0000x reference)
import jax
import jax.numpy as jnp
from jax.experimental import pallas as pl
from jax.experimental.pallas import tpu as pltpu

_N = 20000
_MAX_DET = 200
_SCORE_THRESH = 0.5
_NMS_THRESH = 0.2
_L = 128
_R = 160
_NP = _R * _L
_NEG = -1e9
_IMAX = 2**31 - 1


def _nms_body(x1_ref, y1_ref, x2_ref, y2_ref, sc_ref, out_ref):
    x1 = x1_ref[...]
    y1 = y1_ref[...]
    x2 = x2_ref[...]
    y2 = y2_ref[...]
    area = (x2 - x1) * (y2 - y1)
    s = sc_ref[...]
    sw0 = jnp.where(s > _SCORE_THRESH, s, _NEG)

    rows = jax.lax.broadcasted_iota(jnp.int32, (_R, _L), 0)
    lane1 = jax.lax.broadcasted_iota(jnp.int32, (1, _L), 1)
    lane_sh = lane1 << 16
    row_sh = rows << 23

    def halves(c):
        bits = jax.lax.bitcast_convert_type(c, jnp.int32)
        return row_sh | ((bits >> 16) & 0xFFFF), row_sh | (bits & 0xFFFF)

    keys = [h for c in (x1, y1, x2, y2) for h in halves(c)]

    def winner(sw):
        cm = jnp.max(sw, axis=0, keepdims=True)
        maskc = sw == cm
        colk = [jnp.min(jnp.where(maskc, k, _IMAX), axis=0, keepdims=True)
                for k in keys]
        m = jnp.max(cm, axis=1, keepdims=True)
        lmask = cm == m
        gh = [jnp.min(jnp.where(lmask, ck | lane_sh, _IMAX),
                      axis=1, keepdims=True) for ck in colk]
        coords = []
        for j in range(4):
            g, h = gh[2 * j], gh[2 * j + 1]
            bits = ((g & 0xFFFF) << 16) | (h & 0xFFFF)
            coords.append(jax.lax.bitcast_convert_type(bits, jnp.float32))
        return (m,) + tuple(coords)

    win0 = winner(sw0)

    def body(i, carry):
        sw, bv, w1, w2, w3, w4 = carry
        valid = bv > 0.0
        barea = (w3 - w1) * (w4 - w2)

        xx1 = jnp.maximum(w1, x1)
        yy1 = jnp.maximum(w2, y1)
        xx2 = jnp.minimum(w3, x2)
        yy2 = jnp.minimum(w4, y2)
        inter = jnp.maximum(xx2 - xx1, 0.0) * jnp.maximum(yy2 - yy1, 0.0)
        iou = inter / (barea + area - inter + 1e-9)
        new_sw = jnp.where(jnp.logical_and(valid, iou > _NMS_THRESH), _NEG, sw)

        nwin = winner(new_sw)

        row = jnp.where(
            lane1 == 0, w1,
            jnp.where(lane1 == 1, w2,
                      jnp.where(lane1 == 2, w3,
                                jnp.where(lane1 == 3, w4,
                                          jnp.where(lane1 == 4, bv, 0.0)))))
        row = row * valid.astype(jnp.float32)
        out_ref[pl.ds(i, 1), :] = row
        return (new_sw,) + nwin

    jax.lax.fori_loop(0, _MAX_DET, body, (sw0,) + win0)


def kernel(boxes, scores):
    pad = _NP - _N
    x1 = jnp.pad(boxes[:, 0], (0, pad)).reshape(_R, _L)
    y1 = jnp.pad(boxes[:, 1], (0, pad)).reshape(_R, _L)
    x2 = jnp.pad(boxes[:, 2], (0, pad)).reshape(_R, _L)
    y2 = jnp.pad(boxes[:, 3], (0, pad)).reshape(_R, _L)
    s = jnp.pad(scores, (0, pad)).reshape(_R, _L)

    out = pl.pallas_call(
        _nms_body,
        out_shape=jax.ShapeDtypeStruct((_MAX_DET, _L), jnp.float32),
    )(x1, y1, x2, y2, s)
    return out[:, :5]
